# Initial kernel scaffold; baseline (speedup 1.0000x reference)
import jax
import jax.numpy as jnp
from jax.experimental import pallas as pl


def kernel(inputs, expert_W, expert_b, gate_W, gate_b):
    raise NotImplementedError("write your pallas kernel here")


# speedup vs baseline: 2.3390x; 2.3390x over previous
import jax
import jax.numpy as jnp
from jax.experimental import pallas as pl

_B, _D, _E, _C = 4096, 1024, 8, 1024
_BT = 512


def _moe_body(w_ref, x_ref, W_ref, b_ref, o_ref):
    x = x_ref[...]
    acc = jnp.zeros((_BT, _C), jnp.float32)
    for e in range(_E):
        ye = jnp.dot(x, W_ref[e], preferred_element_type=jnp.float32)
        ye = ye + b_ref[e][None, :]
        acc = acc + w_ref[:, e][:, None] * ye
    o_ref[...] = acc


def kernel(inputs, expert_W, expert_b, gate_W, gate_b):
    logits = inputs @ gate_W + gate_b
    p = jax.nn.softmax(logits, axis=-1)
    sample = jax.random.categorical(jax.random.key(42), jnp.log(p), axis=-1)
    onehot = jax.nn.one_hot(sample, _E, dtype=jnp.float32)

    return pl.pallas_call(
        _moe_body,
        grid=(_B // _BT,),
        in_specs=[
            pl.BlockSpec((_BT, _E), lambda i: (i, 0)),
            pl.BlockSpec((_BT, _D), lambda i: (i, 0)),
            pl.BlockSpec((_E, _D, _C), lambda i: (0, 0, 0)),
            pl.BlockSpec((_E, _C), lambda i: (0, 0)),
        ],
        out_specs=pl.BlockSpec((_BT, _C), lambda i: (i, 0)),
        out_shape=jax.ShapeDtypeStruct((_B, _C), jnp.float32),
    )(onehot, inputs, expert_W, expert_b)
